# Initial kernel scaffold; baseline (speedup 1.0000x reference)
import jax
import jax.numpy as jnp
from jax.experimental import pallas as pl


def kernel(loc, W, bias):
    raise NotImplementedError("write your pallas kernel here")


# speedup vs baseline: 20.0302x; 20.0302x over previous
import functools

import jax
import jax.numpy as jnp
from jax import lax
from jax.experimental import pallas as pl
from jax.experimental.pallas import tpu as pltpu
from jax.experimental.pallas import tpu_sc as plsc

ACTION_SIZE = 1000
BATCH = 16384
LOC_LEN = 5
DIM = 16
NC = 2
NS = 16
NW = NC * NS
BPW = BATCH // NW
ROWS = BPW * LOC_LEN
GCHUNK = 128
NGATHER = ROWS // GCHUNK


def _sc_body(loc_hbm, wt_hbm, bias_hbm, out_hbm, idx_v, rows_v, out_v,
             bias_v, sem):
    wid = lax.axis_index("s") * NC + lax.axis_index("c")
    base = wid * BPW

    pltpu.sync_copy(bias_hbm, bias_v)
    for j in range(LOC_LEN):
        pltpu.sync_copy(loc_hbm.at[pl.ds(j * BATCH + base, BPW)],
                        idx_v.at[pl.ds(j * BPW, BPW)])

    for j in range(1, LOC_LEN):

        def _off_body(c, _, j=j):
            s = j * BPW + c * DIM
            idx_v[pl.ds(s, DIM)] = idx_v[pl.ds(s, DIM)] + j * ACTION_SIZE
            return 0

        lax.fori_loop(0, BPW // DIM, _off_body, 0, unroll=4)

    copies = []
    for t in range(NGATHER):
        copies.append(pltpu.async_copy(
            wt_hbm.at[idx_v.at[pl.ds(t * GCHUNK, GCHUNK)]],
            rows_v.at[pl.ds(t * GCHUNK, GCHUNK), :],
            sem))
    for c in copies:
        c.wait()

    bias_vec = bias_v[...]

    def _sum_body(i, _):
        acc = bias_vec + rows_v[i]
        for j in range(1, LOC_LEN):
            acc = acc + rows_v[j * BPW + i]
        out_v[i] = jnp.maximum(acc, 0.0)
        return 0

    lax.fori_loop(0, BPW, _sum_body, 0, unroll=4)

    pltpu.sync_copy(out_v, out_hbm.at[pl.ds(base, BPW), :])


@jax.jit
def _position_encode(loc_t_flat, wt, bias):
    mesh = plsc.VectorSubcoreMesh(core_axis_name="c", subcore_axis_name="s")
    kern = pl.kernel(
        _sc_body,
        out_type=jax.ShapeDtypeStruct((BATCH, DIM), jnp.float32),
        mesh=mesh,
        scratch_types=[
            pltpu.VMEM((ROWS,), jnp.int32),
            pltpu.VMEM((ROWS, DIM), jnp.float32),
            pltpu.VMEM((BPW, DIM), jnp.float32),
            pltpu.VMEM((DIM,), jnp.float32),
            pltpu.SemaphoreType.DMA,
        ],
        compiler_params=pltpu.CompilerParams(use_tc_tiling_on_sc=False),
    )
    return kern(loc_t_flat, wt, bias)


def kernel(loc, W, bias):
    loc = loc.astype(jnp.int32)
    loc_t_flat = loc.T.reshape(-1)
    wt = W.T.reshape(LOC_LEN * ACTION_SIZE, DIM)
    return _position_encode(loc_t_flat, wt, bias)


# speedup vs baseline: 21.3388x; 1.0653x over previous
import functools

import jax
import jax.numpy as jnp
from jax import lax
from jax.experimental import pallas as pl
from jax.experimental.pallas import tpu as pltpu
from jax.experimental.pallas import tpu_sc as plsc

ACTION_SIZE = 1000
BATCH = 16384
LOC_LEN = 5
DIM = 16
NC = 2
NS = 16
NW = NC * NS
BPW = BATCH // NW
ROWS = BPW * LOC_LEN
GCHUNK = 128
NGATHER = ROWS // GCHUNK


def _sc_body(loc_hbm, wt_hbm, bias_hbm, out_hbm, idx_v, rows_v, out_v,
             bias_v, sem):
    wid = lax.axis_index("s") * NC + lax.axis_index("c")
    base = wid * BPW

    pltpu.sync_copy(bias_hbm, bias_v)
    for j in range(LOC_LEN):
        pltpu.sync_copy(loc_hbm.at[pl.ds(j * BATCH + base, BPW)],
                        idx_v.at[pl.ds(j * BPW, BPW)])

    def _off_body(c, _):
        j = c // (BPW // DIM)
        s = c * DIM
        idx_v[pl.ds(s, DIM)] = idx_v[pl.ds(s, DIM)] + j * ACTION_SIZE
        return 0

    lax.fori_loop(BPW // DIM, ROWS // DIM, _off_body, 0, unroll=4)

    def _g_body(t, _):
        pltpu.async_copy(
            wt_hbm.at[idx_v.at[pl.ds(t * GCHUNK, GCHUNK)]],
            rows_v.at[pl.ds(t * GCHUNK, GCHUNK), :],
            sem)
        return 0

    lax.fori_loop(0, NGATHER, _g_body, 0)
    pltpu.make_async_copy(wt_hbm.at[pl.ds(0, ROWS), :], rows_v, sem).wait()

    bias_vec = bias_v[...]
    col_idx = lax.iota(jnp.int32, DIM) * BPW

    def _sum_body(i, _):
        acc = bias_vec + rows_v[i]
        for j in range(1, LOC_LEN):
            acc = acc + rows_v[j * BPW + i]
        acc = jnp.maximum(acc, 0.0)
        plsc.store_scatter(out_v, [col_idx + i], acc)
        return 0

    lax.fori_loop(0, BPW, _sum_body, 0, unroll=4)

    for k in range(DIM):
        pltpu.sync_copy(out_v.at[pl.ds(k * BPW, BPW)],
                        out_hbm.at[k, pl.ds(base, BPW)])


@jax.jit
def _position_encode(loc_t_flat, wt, bias):
    mesh = plsc.VectorSubcoreMesh(core_axis_name="c", subcore_axis_name="s")
    kern = pl.kernel(
        _sc_body,
        out_type=jax.ShapeDtypeStruct((DIM, BATCH), jnp.float32),
        mesh=mesh,
        scratch_types=[
            pltpu.VMEM((ROWS,), jnp.int32),
            pltpu.VMEM((ROWS, DIM), jnp.float32),
            pltpu.VMEM((DIM * BPW,), jnp.float32),
            pltpu.VMEM((DIM,), jnp.float32),
            pltpu.SemaphoreType.DMA,
        ],
        compiler_params=pltpu.CompilerParams(use_tc_tiling_on_sc=False,
                                             needs_layout_passes=False),
    )
    return kern(loc_t_flat, wt, bias).T


def kernel(loc, W, bias):
    loc = loc.astype(jnp.int32)
    loc_t_flat = loc.T.reshape(-1)
    wt = W.T.reshape(LOC_LEN * ACTION_SIZE, DIM)
    return _position_encode(loc_t_flat, wt, bias)


# speedup vs baseline: 21.8669x; 1.0247x over previous
import functools

import jax
import jax.numpy as jnp
from jax import lax
from jax.experimental import pallas as pl
from jax.experimental.pallas import tpu as pltpu
from jax.experimental.pallas import tpu_sc as plsc

ACTION_SIZE = 1000
BATCH = 16384
LOC_LEN = 5
DIM = 16
NC = 2
NS = 16
NW = NC * NS
BPW = BATCH // NW
ROWS = BPW * LOC_LEN
GCHUNK = 128
NGATHER = ROWS // GCHUNK


def _sc_body(loc_hbm, wt_hbm, bias_hbm, out_hbm, idx_v, rows_v, out_v,
             bias_v, sem):
    wid = lax.axis_index("s") * NC + lax.axis_index("c")
    base = wid * BPW

    pltpu.sync_copy(bias_hbm, bias_v)
    for j in range(LOC_LEN):
        pltpu.sync_copy(loc_hbm.at[pl.ds(j * BATCH + base, BPW)],
                        idx_v.at[pl.ds(j * BPW, BPW)])

    def _off_body(c, _):
        j = c // (BPW // DIM)
        s = c * DIM
        idx_v[pl.ds(s, DIM)] = idx_v[pl.ds(s, DIM)] + j * ACTION_SIZE
        return 0

    lax.fori_loop(BPW // DIM, ROWS // DIM, _off_body, 0, unroll=4)

    def _g_body(t, _):
        pltpu.async_copy(
            wt_hbm.at[idx_v.at[pl.ds(t * GCHUNK, GCHUNK)]],
            rows_v.at[pl.ds(t * GCHUNK, GCHUNK), :],
            sem)
        return 0

    lax.fori_loop(0, NGATHER, _g_body, 0)
    pltpu.make_async_copy(wt_hbm.at[pl.ds(0, ROWS), :], rows_v, sem).wait()

    bias_vec = bias_v[...]
    row_idx = lax.iota(jnp.int32, DIM)
    zeros = jnp.zeros((DIM,), jnp.int32)

    def _sum_body(i, _):
        acc = bias_vec + rows_v[i]
        for j in range(1, LOC_LEN):
            acc = acc + rows_v[j * BPW + i]
        acc = jnp.maximum(acc, 0.0)
        plsc.store_scatter(out_v, [row_idx, zeros + i], acc)
        return 0

    lax.fori_loop(0, BPW, _sum_body, 0, unroll=4)

    pltpu.sync_copy(out_v, out_hbm.at[:, pl.ds(base, BPW)])


@jax.jit
def _position_encode(loc_t_flat, wt, bias):
    mesh = plsc.VectorSubcoreMesh(core_axis_name="c", subcore_axis_name="s")
    kern = pl.kernel(
        _sc_body,
        out_type=jax.ShapeDtypeStruct((DIM, BATCH), jnp.float32),
        mesh=mesh,
        scratch_types=[
            pltpu.VMEM((ROWS,), jnp.int32),
            pltpu.VMEM((ROWS, DIM), jnp.float32),
            pltpu.VMEM((DIM, BPW), jnp.float32),
            pltpu.VMEM((DIM,), jnp.float32),
            pltpu.SemaphoreType.DMA,
        ],
        compiler_params=pltpu.CompilerParams(use_tc_tiling_on_sc=False,
                                             needs_layout_passes=False),
    )
    return kern(loc_t_flat, wt, bias).T


def kernel(loc, W, bias):
    loc = loc.astype(jnp.int32)
    loc_t_flat = loc.T.reshape(-1)
    wt = W.T.reshape(LOC_LEN * ACTION_SIZE, DIM)
    return _position_encode(loc_t_flat, wt, bias)


# speedup vs baseline: 24.1264x; 1.1033x over previous
import functools

import jax
import jax.numpy as jnp
from jax import lax
from jax.experimental import pallas as pl
from jax.experimental.pallas import tpu as pltpu
from jax.experimental.pallas import tpu_sc as plsc

ACTION_SIZE = 1000
BATCH = 16384
LOC_LEN = 5
DIM = 16
NC = 2
NS = 16
NW = NC * NS
BPW = BATCH // NW
ROWS = BPW * LOC_LEN
GCHUNK = 128
NGATHER = ROWS // GCHUNK


def _sc_body(loc_hbm, wt_hbm, bias_hbm, out_hbm, idx_v, rows_v, out_v,
             bias_v, sem):
    wid = lax.axis_index("s") * NC + lax.axis_index("c")
    base = wid * BPW

    pltpu.sync_copy(bias_hbm, bias_v)
    for j in range(LOC_LEN):
        pltpu.sync_copy(loc_hbm.at[pl.ds(j * BATCH + base, BPW)],
                        idx_v.at[pl.ds(j * BPW, BPW)])

    @plsc.parallel_loop(BPW // DIM, ROWS // DIM, 1, unroll=4)
    def _off_body(c):
        j = c // (BPW // DIM)
        s = c * DIM
        idx_v[pl.ds(s, DIM)] = idx_v[pl.ds(s, DIM)] + j * ACTION_SIZE

    def _g_body(t, _):
        pltpu.async_copy(
            wt_hbm.at[idx_v.at[pl.ds(t * GCHUNK, GCHUNK)]],
            rows_v.at[pl.ds(t * GCHUNK, GCHUNK), :],
            sem)
        return 0

    lax.fori_loop(0, NGATHER, _g_body, 0)
    pltpu.make_async_copy(wt_hbm.at[pl.ds(0, ROWS), :], rows_v, sem).wait()

    bias_vec = bias_v[...]
    row_idx = lax.iota(jnp.int32, DIM)
    zeros = jnp.zeros((DIM,), jnp.int32)

    @plsc.parallel_loop(0, BPW, 1, unroll=4)
    def _sum_body(i):
        acc = bias_vec + rows_v[i]
        for j in range(1, LOC_LEN):
            acc = acc + rows_v[j * BPW + i]
        acc = jnp.maximum(acc, 0.0)
        plsc.store_scatter(out_v, [row_idx, zeros + i], acc)

    pltpu.sync_copy(out_v, out_hbm.at[:, pl.ds(base, BPW)])


@jax.jit
def _position_encode(loc_t_flat, wt, bias):
    mesh = plsc.VectorSubcoreMesh(core_axis_name="c", subcore_axis_name="s")
    kern = pl.kernel(
        _sc_body,
        out_type=jax.ShapeDtypeStruct((DIM, BATCH), jnp.float32),
        mesh=mesh,
        scratch_types=[
            pltpu.VMEM((ROWS,), jnp.int32),
            pltpu.VMEM((ROWS, DIM), jnp.float32),
            pltpu.VMEM((DIM, BPW), jnp.float32),
            pltpu.VMEM((DIM,), jnp.float32),
            pltpu.SemaphoreType.DMA,
        ],
        compiler_params=pltpu.CompilerParams(use_tc_tiling_on_sc=False,
                                             needs_layout_passes=False),
    )
    return kern(loc_t_flat, wt, bias).T


def kernel(loc, W, bias):
    loc = loc.astype(jnp.int32)
    loc_t_flat = loc.T.reshape(-1)
    wt = W.T.reshape(LOC_LEN * ACTION_SIZE, DIM)
    return _position_encode(loc_t_flat, wt, bias)


# speedup vs baseline: 24.5532x; 1.0177x over previous
import functools

import jax
import jax.numpy as jnp
from jax import lax
from jax.experimental import pallas as pl
from jax.experimental.pallas import tpu as pltpu
from jax.experimental.pallas import tpu_sc as plsc

ACTION_SIZE = 1000
BATCH = 16384
LOC_LEN = 5
DIM = 16
NC = 2
NS = 16
NW = NC * NS
BPW = BATCH // NW
ROWS = BPW * LOC_LEN
GCHUNK = 128
NGATHER = ROWS // GCHUNK


def _sc_body(loc_hbm, wt_hbm, bias_hbm, out_hbm, idx_v, rows_v, out_v,
             bias_v, sem):
    wid = lax.axis_index("s") * NC + lax.axis_index("c")
    base = wid * BPW

    pltpu.sync_copy(bias_hbm, bias_v)
    for j in range(LOC_LEN):
        pltpu.sync_copy(loc_hbm.at[pl.ds(j * BATCH + base, BPW)],
                        idx_v.at[pl.ds(j * BPW, BPW)])

    @plsc.parallel_loop(BPW // DIM, ROWS // DIM, 1, unroll=4)
    def _off_body(c):
        j = c // (BPW // DIM)
        s = c * DIM
        idx_v[pl.ds(s, DIM)] = idx_v[pl.ds(s, DIM)] + j * ACTION_SIZE

    bias_vec = bias_v[...]

    @plsc.parallel_loop(0, BPW, 1, unroll=8)
    def _fill_body(i):
        rows_v[i] = bias_vec

    def _g_body(t, _):
        pltpu.async_copy(
            wt_hbm.at[idx_v.at[pl.ds(t * GCHUNK, GCHUNK)]],
            rows_v.at[pl.ds(lax.rem(t, BPW // GCHUNK) * GCHUNK, GCHUNK), :],
            sem, add=True)
        return 0

    lax.fori_loop(0, NGATHER, _g_body, 0)

    def _d_body(t, _):
        pltpu.make_async_copy(wt_hbm.at[pl.ds(0, GCHUNK), :],
                              rows_v.at[pl.ds(0, GCHUNK), :], sem).wait()
        return 0

    lax.fori_loop(0, NGATHER, _d_body, 0)

    row_idx = lax.iota(jnp.int32, DIM)
    zeros = jnp.zeros((DIM,), jnp.int32)

    @plsc.parallel_loop(0, BPW, 1, unroll=8)
    def _sum_body(i):
        acc = jnp.maximum(rows_v[i], 0.0)
        plsc.store_scatter(out_v, [row_idx, zeros + i], acc)

    pltpu.sync_copy(out_v, out_hbm.at[:, pl.ds(base, BPW)])


@jax.jit
def _position_encode(loc_t_flat, wt, bias):
    mesh = plsc.VectorSubcoreMesh(core_axis_name="c", subcore_axis_name="s")
    kern = pl.kernel(
        _sc_body,
        out_type=jax.ShapeDtypeStruct((DIM, BATCH), jnp.float32),
        mesh=mesh,
        scratch_types=[
            pltpu.VMEM((ROWS,), jnp.int32),
            pltpu.VMEM((BPW, DIM), jnp.float32),
            pltpu.VMEM((DIM, BPW), jnp.float32),
            pltpu.VMEM((DIM,), jnp.float32),
            pltpu.SemaphoreType.DMA,
        ],
        compiler_params=pltpu.CompilerParams(use_tc_tiling_on_sc=False,
                                             needs_layout_passes=False),
    )
    return kern(loc_t_flat, wt, bias).T


def kernel(loc, W, bias):
    loc = loc.astype(jnp.int32)
    loc_t_flat = loc.T.reshape(-1)
    wt = W.T.reshape(LOC_LEN * ACTION_SIZE, DIM)
    return _position_encode(loc_t_flat, wt, bias)
